# Initial kernel scaffold; baseline (speedup 1.0000x reference)
import jax
import jax.numpy as jnp
from jax.experimental import pallas as pl


def kernel(batch, unigrams, bigrams, trigrams_tab, w):
    raise NotImplementedError("write your pallas kernel here")


# speedup vs baseline: 43.8035x; 43.8035x over previous
import functools

import jax
import jax.numpy as jnp
from jax import lax
from jax.experimental import pallas as pl
from jax.experimental.pallas import tpu as pltpu
from jax.experimental.pallas import tpu_sc as plsc

V = 256
B = 65536
P = B
N_SUM = B - 1
NC = 2
NS = 16
NW = NC * NS
C = P // NW
L = 16
CHUNK = 128


def _rowsum_body(tri_ref, big_ref, rs2_ref, rs1_ref):
    rs2_ref[...] = jnp.sum(tri_ref[...], axis=-1)

    @pl.when(pl.program_id(0) == 0)
    def _():
        rs1_ref[...] = jnp.sum(big_ref[...], axis=1, keepdims=True)


def _rowsums(trigrams_tab, bigrams):
    rblk = 8
    return pl.pallas_call(
        _rowsum_body,
        grid=(V // rblk,),
        in_specs=[
            pl.BlockSpec((rblk, V, V), lambda g: (g, 0, 0)),
            pl.BlockSpec((V, V), lambda g: (0, 0)),
        ],
        out_specs=[
            pl.BlockSpec((rblk, V), lambda g: (g, 0)),
            pl.BlockSpec((V, 1), lambda g: (0, 0)),
        ],
        out_shape=[
            jax.ShapeDtypeStruct((V, V), jnp.float32),
            jax.ShapeDtypeStruct((V, 1), jnp.float32),
        ],
    )(trigrams_tab, bigrams)


def _gather_body(bpad_hbm, uni_hbm, big_hbm, tri_hbm, rs1_hbm, rs2_hbm,
                 g1_hbm, g2_hbm, g3_hbm, ps_hbm,
                 bvec, univ, rs1v, rs2v, tri_idx, bi_idx,
                 g1b, g2b, g3b, psb, sem):
    wid = lax.axis_index("s") * NC + lax.axis_index("c")
    base = wid * C

    pltpu.sync_copy(bpad_hbm.at[pl.ds(base, C + 8)], bvec)
    pltpu.sync_copy(uni_hbm, univ)
    pltpu.sync_copy(rs1_hbm, rs1v)
    pltpu.sync_copy(rs2_hbm, rs2v)

    lane = lax.iota(jnp.int32, L)

    def step(j, carry):
        ps1, ps2 = carry
        off = j * L
        b0 = bvec[pl.ds(off, L)]
        b1 = plsc.load_gather(bvec, [lane + (off + 1)])
        b2 = plsc.load_gather(bvec, [lane + (off + 2)])
        pair = b0 * V + b1
        tri_idx[pl.ds(off, L)] = pair * V + b2
        bi_idx[pl.ds(off, L)] = b1 * V + b2
        g1b[pl.ds(off, L)] = plsc.load_gather(univ, [b2])
        rs1g = plsc.load_gather(rs1v, [b1])
        rs2g = plsc.load_gather(rs2v, [pair])
        m = (base + off + lane) < N_SUM
        ps1 = ps1 + jnp.where(m, rs1g, 0.0)
        ps2 = ps2 + jnp.where(m, rs2g, 0.0)
        return ps1, ps2

    zero0 = jnp.zeros((L,), jnp.float32)
    ps1, ps2 = lax.fori_loop(0, C // L, step, (zero0, zero0))

    copies = []
    for k in range(C // CHUNK):
        sl = pl.ds(k * CHUNK, CHUNK)
        copies.append(pltpu.async_copy(tri_hbm.at[tri_idx.at[sl]], g3b.at[sl], sem))
        copies.append(pltpu.async_copy(big_hbm.at[bi_idx.at[sl]], g2b.at[sl], sem))
    for cp in copies:
        cp.wait()

    zero = jnp.zeros((L,), jnp.float32)
    for t in range(128 // L):
        psb[pl.ds(t * L, L)] = zero
    psb[pl.ds(0, L)] = ps1
    psb[pl.ds(L, L)] = ps2
    out_sl = pl.ds(base, C)
    pltpu.sync_copy(g1b, g1_hbm.at[out_sl])
    pltpu.sync_copy(g2b, g2_hbm.at[out_sl])
    pltpu.sync_copy(g3b, g3_hbm.at[out_sl])
    pltpu.sync_copy(psb, ps_hbm.at[wid])


def _sc_gather(bpad, unigrams, big_flat, tri_flat, rs1, rs2_flat):
    mesh = plsc.VectorSubcoreMesh(core_axis_name="c", subcore_axis_name="s",
                                  num_cores=NC, num_subcores=NS)
    f32 = jnp.float32
    call = pl.kernel(
        _gather_body,
        out_type=[
            jax.ShapeDtypeStruct((P,), f32),
            jax.ShapeDtypeStruct((P,), f32),
            jax.ShapeDtypeStruct((P,), f32),
            jax.ShapeDtypeStruct((NW, 128), f32),
        ],
        mesh=mesh,
        compiler_params=pltpu.CompilerParams(needs_layout_passes=False),
        scratch_types=[
            pltpu.VMEM((C + 8,), jnp.int32),
            pltpu.VMEM((V,), f32),
            pltpu.VMEM((V,), f32),
            pltpu.VMEM((V * V,), f32),
            pltpu.VMEM((C,), jnp.int32),
            pltpu.VMEM((C,), jnp.int32),
            pltpu.VMEM((C,), f32),
            pltpu.VMEM((C,), f32),
            pltpu.VMEM((C,), f32),
            pltpu.VMEM((128,), f32),
            pltpu.SemaphoreType.DMA,
        ],
    )
    return call(bpad, unigrams, big_flat, tri_flat, rs1, rs2_flat)


def _combine_body(w_ref, ps_ref, uni_ref, g1_ref, g2_ref, g3_ref,
                  o_ref):
    usum = jnp.sum(uni_ref[...])
    ps = ps_ref[...]
    s1 = jnp.sum(ps[:, 0:L])
    s2 = jnp.sum(ps[:, L:2 * L])
    w0 = w_ref[0, 0]
    w1 = w_ref[0, 1]
    w2 = w_ref[0, 2]
    w3 = w_ref[0, 3]

    def rb(x):
        return x.astype(jnp.bfloat16).astype(jnp.float32)

    p1 = rb(g1_ref[...] / usum)
    p2 = rb(g2_ref[...] / s1)
    p3 = rb(g3_ref[...] / s2)
    o_ref[...] = (w0 * (1.0 / V)) + p1 * w1 + p2 * w2 + p3 * w3


def _combine(w, ps, uni2d, g1, g2, g3):
    rows = P // 128
    return pl.pallas_call(
        _combine_body,
        in_specs=[
            pl.BlockSpec(memory_space=pltpu.SMEM),
            pl.BlockSpec((NW, 128), lambda: (0, 0)),
            pl.BlockSpec((2, 128), lambda: (0, 0)),
            pl.BlockSpec((rows, 128), lambda: (0, 0)),
            pl.BlockSpec((rows, 128), lambda: (0, 0)),
            pl.BlockSpec((rows, 128), lambda: (0, 0)),
        ],
        out_specs=pl.BlockSpec((rows, 128), lambda: (0, 0)),
        out_shape=jax.ShapeDtypeStruct((rows, 128), jnp.float32),
    )(w, ps, uni2d, g1, g2, g3)


def kernel(batch, unigrams, bigrams, trigrams_tab, w):
    rs2, rs1 = _rowsums(trigrams_tab, bigrams)
    bpad = jnp.concatenate([batch, jnp.zeros((8,), jnp.int32)])
    g1, g2, g3, ps = _sc_gather(
        bpad,
        unigrams,
        bigrams.reshape(V * V),
        trigrams_tab.reshape(V * V * V),
        rs1.reshape(V),
        rs2.reshape(V * V),
    )
    rows = P // 128
    out = _combine(
        w.astype(jnp.bfloat16).astype(jnp.float32), ps,
        unigrams.reshape(2, 128),
        g1.reshape(rows, 128),
        g2.reshape(rows, 128),
        g3.reshape(rows, 128),
    )
    return out.reshape(P)[: B - 2]
